# baseline (speedup 1.0000x reference)
import jax
import jax.numpy as jnp
from jax import lax
from jax.experimental import pallas as pl
from jax.experimental.pallas import tpu as pltpu
from jax.experimental.pallas import tpu_sc as plsc

N_ATOMS = 100000
LANES = 16
NUM_CORES = 1
NUM_WORKERS = 16 * NUM_CORES
MAIN = (N_ATOMS // NUM_WORKERS) // LANES * LANES
TAIL = N_ATOMS - MAIN * NUM_WORKERS
BUF = MAIN + TAIL
SQRT_SHIFT = 1e-6
REF_VALUE = 1.0


def _shift_vec(cn_vec, kap_vec):
    x = cn_vec + SQRT_SHIFT
    xi = plsc.bitcast(x, jnp.int32)
    yi = jnp.int32(0x5F3759DF) - lax.shift_right_logical(xi, 1)
    y = plsc.bitcast(yi, jnp.float32)
    half_x = 0.5 * x
    for _ in range(2):
        y = y * (1.5 - half_x * y * y)
    return REF_VALUE + kap_vec * (x * y)


HALF = MAIN // 2


def _body(cn_hbm, sp_hbm, kap_hbm, out_hbm, cn_v, sp_v, out_v, kap_v, s0, s1, so, st):
    wid = lax.axis_index("s") * NUM_CORES + lax.axis_index("c")
    base = wid * MAIN
    is_last = wid == NUM_WORKERS - 1

    a1 = pltpu.async_copy(kap_hbm, kap_v, s0)
    a2 = pltpu.async_copy(cn_hbm.at[pl.ds(base, HALF)], cn_v.at[pl.ds(0, HALF)], s0)
    a3 = pltpu.async_copy(sp_hbm.at[pl.ds(base, HALF)], sp_v.at[pl.ds(0, HALF)], s0)
    b1 = pltpu.async_copy(
        cn_hbm.at[pl.ds(base + HALF, HALF)], cn_v.at[pl.ds(HALF, HALF)], s1
    )
    b2 = pltpu.async_copy(
        sp_hbm.at[pl.ds(base + HALF, HALF)], sp_v.at[pl.ds(HALF, HALF)], s1
    )

    @pl.when(is_last)
    def _tail_in():
        pltpu.async_copy(
            cn_hbm.at[pl.ds(MAIN * NUM_WORKERS, TAIL)], cn_v.at[pl.ds(MAIN, TAIL)], st
        )
        pltpu.async_copy(
            sp_hbm.at[pl.ds(MAIN * NUM_WORKERS, TAIL)], sp_v.at[pl.ds(MAIN, TAIL)], st
        )

    def _compute(off):
        cn_vec = cn_v[pl.ds(off, LANES)]
        sp_vec = sp_v[pl.ds(off, LANES)]
        kap_vec = plsc.load_gather(kap_v, [sp_vec])
        out_v[pl.ds(off, LANES)] = _shift_vec(cn_vec, kap_vec)

    a1.wait()
    a2.wait()
    a3.wait()
    plsc.parallel_loop(0, HALF, step=LANES, unroll=8)(_compute)
    o1 = pltpu.async_copy(out_v.at[pl.ds(0, HALF)], out_hbm.at[pl.ds(base, HALF)], so)

    b1.wait()
    b2.wait()
    plsc.parallel_loop(HALF, MAIN, step=LANES, unroll=8)(_compute)
    o2 = pltpu.async_copy(
        out_v.at[pl.ds(HALF, HALF)], out_hbm.at[pl.ds(base + HALF, HALF)], so
    )

    @pl.when(is_last)
    def _tail_out():
        pltpu.make_async_copy(
            cn_hbm.at[pl.ds(MAIN * NUM_WORKERS, TAIL)], cn_v.at[pl.ds(MAIN, TAIL)], st
        ).wait()
        pltpu.make_async_copy(
            sp_hbm.at[pl.ds(MAIN * NUM_WORKERS, TAIL)], sp_v.at[pl.ds(MAIN, TAIL)], st
        ).wait()
        plsc.parallel_loop(MAIN, BUF, step=LANES, unroll=2)(_compute)
        pltpu.async_copy(
            out_v.at[pl.ds(MAIN, TAIL)],
            out_hbm.at[pl.ds(MAIN * NUM_WORKERS, TAIL)],
            so,
        ).wait()

    o1.wait()
    o2.wait()


@jax.jit
def _cnshift_sc(cn, species, kappa):
    mesh = plsc.VectorSubcoreMesh(
        core_axis_name="c", subcore_axis_name="s", num_cores=NUM_CORES
    )
    return pl.kernel(
        _body,
        mesh=mesh,
        out_type=jax.ShapeDtypeStruct((N_ATOMS,), jnp.float32),
        compiler_params=pltpu.CompilerParams(needs_layout_passes=False),
        scratch_types=[
            pltpu.VMEM((BUF,), jnp.float32),
            pltpu.VMEM((BUF,), jnp.int32),
            pltpu.VMEM((BUF,), jnp.float32),
            pltpu.VMEM((94,), jnp.float32),
            pltpu.SemaphoreType.DMA,
            pltpu.SemaphoreType.DMA,
            pltpu.SemaphoreType.DMA,
            pltpu.SemaphoreType.DMA,
        ],
    )(cn, species, kappa)


def kernel(cn, species, kappa):
    return _cnshift_sc(cn, species, kappa)


# speedup vs baseline: 1.0179x; 1.0179x over previous
import jax
import jax.numpy as jnp
from jax import lax
from jax.experimental import pallas as pl
from jax.experimental.pallas import tpu as pltpu
from jax.experimental.pallas import tpu_sc as plsc

N_ATOMS = 100000
LANES = 16
NUM_CORES = 1
NUM_WORKERS = 16 * NUM_CORES
MAIN = (N_ATOMS // NUM_WORKERS) // LANES * LANES
TAIL = N_ATOMS - MAIN * NUM_WORKERS
BUF = MAIN + TAIL
SQRT_SHIFT = 1e-6
REF_VALUE = 1.0


def _shift_vec(cn_vec, kap_vec):
    x = cn_vec + SQRT_SHIFT
    xi = plsc.bitcast(x, jnp.int32)
    yi = jnp.int32(0x5F3759DF) - lax.shift_right_logical(xi, 1)
    y = plsc.bitcast(yi, jnp.float32)
    half_x = 0.5 * x
    for _ in range(2):
        y = y * (1.5 - half_x * y * y)
    return REF_VALUE + kap_vec * (x * y)


def _body(cn_hbm, sp_hbm, kap_hbm, out_hbm, cn_v, sp_v, out_v, kap_v, sem):
    wid = lax.axis_index("s") * NUM_CORES + lax.axis_index("c")
    base = wid * MAIN
    is_last = wid == NUM_WORKERS - 1

    c1 = pltpu.async_copy(kap_hbm, kap_v, sem)
    c2 = pltpu.async_copy(cn_hbm.at[pl.ds(base, MAIN)], cn_v.at[pl.ds(0, MAIN)], sem)
    c3 = pltpu.async_copy(sp_hbm.at[pl.ds(base, MAIN)], sp_v.at[pl.ds(0, MAIN)], sem)

    @pl.when(is_last)
    def _tail_in():
        t1 = pltpu.async_copy(
            cn_hbm.at[pl.ds(MAIN * NUM_WORKERS, TAIL)], cn_v.at[pl.ds(MAIN, TAIL)], sem
        )
        t2 = pltpu.async_copy(
            sp_hbm.at[pl.ds(MAIN * NUM_WORKERS, TAIL)], sp_v.at[pl.ds(MAIN, TAIL)], sem
        )
        t1.wait()
        t2.wait()

    c1.wait()
    c2.wait()
    c3.wait()

    def _compute(off):
        cn_vec = cn_v[pl.ds(off, LANES)]
        sp_vec = sp_v[pl.ds(off, LANES)]
        kap_vec = plsc.load_gather(kap_v, [sp_vec])
        out_v[pl.ds(off, LANES)] = _shift_vec(cn_vec, kap_vec)

    plsc.parallel_loop(0, MAIN, step=LANES, unroll=8)(_compute)

    o1 = pltpu.async_copy(out_v.at[pl.ds(0, MAIN)], out_hbm.at[pl.ds(base, MAIN)], sem)

    @pl.when(is_last)
    def _tail_out():
        plsc.parallel_loop(MAIN, BUF, step=LANES, unroll=2)(_compute)
        pltpu.async_copy(
            out_v.at[pl.ds(MAIN, TAIL)],
            out_hbm.at[pl.ds(MAIN * NUM_WORKERS, TAIL)],
            sem,
        ).wait()

    o1.wait()


@jax.jit
def _cnshift_sc(cn, species, kappa):
    mesh = plsc.VectorSubcoreMesh(
        core_axis_name="c", subcore_axis_name="s", num_cores=NUM_CORES
    )
    return pl.kernel(
        _body,
        mesh=mesh,
        out_type=jax.ShapeDtypeStruct((N_ATOMS,), jnp.float32),
        compiler_params=pltpu.CompilerParams(needs_layout_passes=False),
        scratch_types=[
            pltpu.VMEM((BUF,), jnp.float32),
            pltpu.VMEM((BUF,), jnp.int32),
            pltpu.VMEM((BUF,), jnp.float32),
            pltpu.VMEM((94,), jnp.float32),
            pltpu.SemaphoreType.DMA,
        ],
    )(cn, species, kappa)


def kernel(cn, species, kappa):
    return _cnshift_sc(cn, species, kappa)


# speedup vs baseline: 1.0277x; 1.0096x over previous
import jax
import jax.numpy as jnp
from jax import lax
from jax.experimental import pallas as pl
from jax.experimental.pallas import tpu as pltpu
from jax.experimental.pallas import tpu_sc as plsc

N_ATOMS = 100000
LANES = 16
NUM_CORES = 1
NUM_WORKERS = 16 * NUM_CORES
MAIN = (N_ATOMS // NUM_WORKERS) // LANES * LANES
TAIL = N_ATOMS - MAIN * NUM_WORKERS
BUF = MAIN + TAIL
SQRT_SHIFT = 1e-6
REF_VALUE = 1.0


def _shift_vec(cn_vec, kap_vec):
    x = cn_vec + SQRT_SHIFT
    xi = plsc.bitcast(x, jnp.int32)
    yi = jnp.int32(0x5F3759DF) - lax.shift_right_logical(xi, 1)
    y = plsc.bitcast(yi, jnp.float32)
    half_x = 0.5 * x
    for _ in range(2):
        y = y * (1.5 - half_x * y * y)
    return REF_VALUE + kap_vec * (x * y)


def _body(cn_hbm, sp_hbm, kap_hbm, out_hbm, cn_v, sp_v, out_v, kap_v, sem):
    wid = lax.axis_index("s") * NUM_CORES + lax.axis_index("c")
    base = wid * MAIN
    is_last = wid == NUM_WORKERS - 1

    c1 = pltpu.async_copy(kap_hbm, kap_v, sem)
    c2 = pltpu.async_copy(cn_hbm.at[pl.ds(base, MAIN)], cn_v.at[pl.ds(0, MAIN)], sem)
    c3 = pltpu.async_copy(sp_hbm.at[pl.ds(base, MAIN)], sp_v.at[pl.ds(0, MAIN)], sem)

    @pl.when(is_last)
    def _tail_in():
        t1 = pltpu.async_copy(
            cn_hbm.at[pl.ds(MAIN * NUM_WORKERS, TAIL)], cn_v.at[pl.ds(MAIN, TAIL)], sem
        )
        t2 = pltpu.async_copy(
            sp_hbm.at[pl.ds(MAIN * NUM_WORKERS, TAIL)], sp_v.at[pl.ds(MAIN, TAIL)], sem
        )
        t1.wait()
        t2.wait()

    c1.wait()
    c2.wait()
    c3.wait()

    def _compute(off):
        cn_vec = cn_v[pl.ds(off, LANES)]
        sp_vec = sp_v[pl.ds(off, LANES)]
        kap_vec = plsc.load_gather(kap_v, [sp_vec])
        out_v[pl.ds(off, LANES)] = _shift_vec(cn_vec, kap_vec)

    plsc.parallel_loop(0, MAIN, step=LANES, unroll=8)(_compute)

    o1 = pltpu.async_copy(out_v.at[pl.ds(0, MAIN)], out_hbm.at[pl.ds(base, MAIN)], sem)

    @pl.when(is_last)
    def _tail_out():
        plsc.parallel_loop(MAIN, BUF, step=LANES, unroll=2)(_compute)
        pltpu.async_copy(
            out_v.at[pl.ds(MAIN, TAIL)],
            out_hbm.at[pl.ds(MAIN * NUM_WORKERS, TAIL)],
            sem,
        ).wait()

    o1.wait()


@jax.jit
def _cnshift_sc(cn, species, kappa):
    mesh = plsc.VectorSubcoreMesh(
        core_axis_name="c", subcore_axis_name="s", num_cores=NUM_CORES
    )
    return pl.kernel(
        _body,
        mesh=mesh,
        out_type=jax.ShapeDtypeStruct((N_ATOMS,), jnp.float32),
        compiler_params=pltpu.CompilerParams(
            needs_layout_passes=False, use_tc_tiling_on_sc=False
        ),
        scratch_types=[
            pltpu.VMEM((BUF,), jnp.float32),
            pltpu.VMEM((BUF,), jnp.int32),
            pltpu.VMEM((BUF,), jnp.float32),
            pltpu.VMEM((94,), jnp.float32),
            pltpu.SemaphoreType.DMA,
        ],
    )(cn, species, kappa)


def kernel(cn, species, kappa):
    return _cnshift_sc(cn, species, kappa)
